# baseline (speedup 1.0000x reference)
import jax
import jax.numpy as jnp
from jax import lax
from jax.experimental import pallas as pl
from jax.experimental.pallas import tpu as pltpu
from jax.experimental.pallas import tpu_sc as plsc

NROWS = 128
NCOLS = 8192
LANES = 16
NVREG = NCOLS // LANES
NC = 2
NS = 16
NW = NC * NS
ROWS_PER_W = NROWS // NW
EPS = 1e-10


def _vget(ref, vi):
    return ref[pl.ds(vi * LANES, LANES)]


def _vput(ref, vi, x):
    ref[pl.ds(vi * LANES, LANES)] = x


def _sc_sort_body(al_hbm, sc_hbm, out_hbm, key_v, val_v):
    wid = lax.axis_index("s") * NC + lax.axis_index("c")

    for rr in range(ROWS_PER_W):
        base = (wid * ROWS_PER_W + rr) * NCOLS
        pltpu.sync_copy(al_hbm.at[pl.ds(base, NCOLS)], key_v)
        pltpu.sync_copy(sc_hbm.at[pl.ds(base, NCOLS)], val_v)

        def init_body(i, _):
            a = _vget(key_v, i)
            bad = (a != a) | (jnp.abs(a) == jnp.inf)
            k = -jnp.where(bad, 0.0, a)
            ks, vs = plsc.sort_key_val(k, _vget(val_v, i))
            _vput(key_v, i, ks)
            _vput(val_v, i, vs)
            return 0

        lax.fori_loop(0, NVREG, init_body, 0)

        for kblk in (32, 64, 128, 256, 512, 1024, 2048, 4096, 8192):
            nv = kblk // LANES
            half = nv // 2

            def reflect_body(p, _, nv=nv, half=half):
                blk = p // half
                j = p - blk * half
                a_i = blk * nv + j
                b_i = blk * nv + (nv - 1 - j)
                ka = _vget(key_v, a_i)
                va = _vget(val_v, a_i)
                kb = lax.rev(_vget(key_v, b_i), (0,))
                vb = lax.rev(_vget(val_v, b_i), (0,))
                cm = ka <= kb
                _vput(key_v, a_i, jnp.where(cm, ka, kb))
                _vput(val_v, a_i, jnp.where(cm, va, vb))
                _vput(key_v, b_i, lax.rev(jnp.where(cm, kb, ka), (0,)))
                _vput(val_v, b_i, lax.rev(jnp.where(cm, vb, va), (0,)))
                return 0

            lax.fori_loop(0, NVREG // 2, reflect_body, 0)

            d = half // 2
            while d >= 1:

                def halv_body(p, _, d=d):
                    q = p // d
                    a_i = q * (2 * d) + (p - q * d)
                    b_i = a_i + d
                    ka = _vget(key_v, a_i)
                    va = _vget(val_v, a_i)
                    kb = _vget(key_v, b_i)
                    vb = _vget(val_v, b_i)
                    cm = ka <= kb
                    _vput(key_v, a_i, jnp.where(cm, ka, kb))
                    _vput(val_v, a_i, jnp.where(cm, va, vb))
                    _vput(key_v, b_i, jnp.where(cm, kb, ka))
                    _vput(val_v, b_i, jnp.where(cm, vb, va))
                    return 0

                lax.fori_loop(0, NVREG // 2, halv_body, 0)
                d //= 2

            def sort_body(i, _):
                ks, vs = plsc.sort_key_val(_vget(key_v, i), _vget(val_v, i))
                _vput(key_v, i, ks)
                _vput(val_v, i, vs)
                return 0

            lax.fori_loop(0, NVREG, sort_body, 0)

        pltpu.sync_copy(val_v, out_hbm.at[pl.ds(base, NCOLS)])


def _sc_sort(al_flat, sc_flat):
    mesh = plsc.VectorSubcoreMesh(core_axis_name="c", subcore_axis_name="s")
    f = pl.kernel(
        _sc_sort_body,
        out_type=jax.ShapeDtypeStruct((NROWS * NCOLS,), jnp.float32),
        mesh=mesh,
        scratch_types=[
            pltpu.VMEM((NCOLS,), jnp.float32),
            pltpu.VMEM((NCOLS,), jnp.float32),
        ],
        compiler_params=pltpu.CompilerParams(needs_layout_passes=False),
    )
    return f(al_flat, sc_flat)


_RB = 32


def _tc_loss_body(y_ref, out_ref):
    i = pl.program_id(0)
    y = y_ref[...]
    s = jnp.where(jnp.isnan(y) | jnp.isinf(y), 0.0, y)
    s = jnp.clip(s, -50.0, 50.0)
    m = jnp.max(s, axis=1, keepdims=True)
    e = jnp.exp(s - m).reshape(_RB * 64, 128)
    li = lax.broadcasted_iota(jnp.int32, (128, 128), 0)
    lj = lax.broadcasted_iota(jnp.int32, (128, 128), 1)
    tl = (li >= lj).astype(jnp.float32)
    w = lax.dot_general(
        e, tl, (((1,), (0,)), ((), ())),
        preferred_element_type=jnp.float32,
        precision=lax.Precision.HIGHEST,
    ).reshape(_RB, 64, 128)
    bs = jnp.sum(e.reshape(_RB, 64, 128), axis=2)
    bi = lax.broadcasted_iota(jnp.int32, (64, 64), 0)
    bj = lax.broadcasted_iota(jnp.int32, (64, 64), 1)
    tb = (bi > bj).astype(jnp.float32)
    sb = lax.dot_general(
        bs, tb, (((1,), (0,)), ((), ())),
        preferred_element_type=jnp.float32,
        precision=lax.Precision.HIGHEST,
    )
    suf = w + sb[:, :, None]
    logc = jnp.log(suf + EPS) + m[:, :, None]
    part = (jnp.sum(logc) - jnp.sum(s)) / (NROWS * NCOLS)

    @pl.when(i == 0)
    def _():
        out_ref[...] = jnp.zeros((1, 1), jnp.float32)

    out_ref[...] += jnp.reshape(part, (1, 1))


def _tc_loss(sorted_scores):
    out = pl.pallas_call(
        _tc_loss_body,
        grid=(NROWS // _RB,),
        in_specs=[pl.BlockSpec((_RB, NCOLS), lambda i: (i, 0))],
        out_specs=pl.BlockSpec((1, 1), lambda i: (0, 0)),
        out_shape=jax.ShapeDtypeStruct((1, 1), jnp.float32),
    )(sorted_scores)
    return out[0, 0]


def kernel(scores, auxiliary_labels):
    s = scores.astype(jnp.float32).reshape(-1)
    al = auxiliary_labels.astype(jnp.float32).reshape(-1)
    sorted_flat = _sc_sort(al, s)
    return _tc_loss(sorted_flat.reshape(NROWS, NCOLS))


# speedup vs baseline: 2.1480x; 2.1480x over previous
import jax
import jax.numpy as jnp
from jax import lax
from jax.experimental import pallas as pl
from jax.experimental.pallas import tpu as pltpu
from jax.experimental.pallas import tpu_sc as plsc

NROWS = 128
NCOLS = 8192
LANES = 16
NVREG = NCOLS // LANES
NC = 2
NS = 16
NW = NC * NS
ROWS_PER_W = NROWS // NW
EPS = 1e-10


def _vget(ref, vi):
    return ref[pl.ds(vi * LANES, LANES)]


def _vput(ref, vi, x):
    ref[pl.ds(vi * LANES, LANES)] = x


def _sc_sort_body(al_hbm, sc_hbm, out_hbm, key_v, val_v):
    wid = lax.axis_index("s") * NC + lax.axis_index("c")

    for rr in range(ROWS_PER_W):
        base = (wid * ROWS_PER_W + rr) * NCOLS
        pltpu.sync_copy(al_hbm.at[pl.ds(base, NCOLS)], key_v)
        pltpu.sync_copy(sc_hbm.at[pl.ds(base, NCOLS)], val_v)

        def init_body(i):
            a = _vget(key_v, i)
            bad = (a != a) | (jnp.abs(a) == jnp.inf)
            k = -jnp.where(bad, 0.0, a)
            ks, vs = plsc.sort_key_val(k, _vget(val_v, i))
            _vput(key_v, i, ks)
            _vput(val_v, i, vs)

        plsc.parallel_loop(0, NVREG)(init_body)

        for kblk in (32, 64, 128, 256, 512, 1024, 2048, 4096, 8192):
            nv = kblk // LANES
            half = nv // 2

            def reflect_body(p, nv=nv, half=half):
                blk = p // half
                j = p - blk * half
                a_i = blk * nv + j
                b_i = blk * nv + (nv - 1 - j)
                ka = _vget(key_v, a_i)
                va = _vget(val_v, a_i)
                kb = lax.rev(_vget(key_v, b_i), (0,))
                vb = lax.rev(_vget(val_v, b_i), (0,))
                cm = ka <= kb
                _vput(key_v, a_i, jnp.where(cm, ka, kb))
                _vput(val_v, a_i, jnp.where(cm, va, vb))
                _vput(key_v, b_i, lax.rev(jnp.where(cm, kb, ka), (0,)))
                _vput(val_v, b_i, lax.rev(jnp.where(cm, vb, va), (0,)))

            plsc.parallel_loop(0, NVREG // 2)(reflect_body)

            d = half // 2
            while d >= 1:

                def halv_body(p, d=d):
                    q = p // d
                    a_i = q * (2 * d) + (p - q * d)
                    b_i = a_i + d
                    ka = _vget(key_v, a_i)
                    va = _vget(val_v, a_i)
                    kb = _vget(key_v, b_i)
                    vb = _vget(val_v, b_i)
                    cm = ka <= kb
                    _vput(key_v, a_i, jnp.where(cm, ka, kb))
                    _vput(val_v, a_i, jnp.where(cm, va, vb))
                    _vput(key_v, b_i, jnp.where(cm, kb, ka))
                    _vput(val_v, b_i, jnp.where(cm, vb, va))

                plsc.parallel_loop(0, NVREG // 2)(halv_body)
                d //= 2

            def sort_body(i):
                ks, vs = plsc.sort_key_val(_vget(key_v, i), _vget(val_v, i))
                _vput(key_v, i, ks)
                _vput(val_v, i, vs)

            plsc.parallel_loop(0, NVREG)(sort_body)

        pltpu.sync_copy(val_v, out_hbm.at[pl.ds(base, NCOLS)])


def _sc_sort(al_flat, sc_flat):
    mesh = plsc.VectorSubcoreMesh(core_axis_name="c", subcore_axis_name="s")
    f = pl.kernel(
        _sc_sort_body,
        out_type=jax.ShapeDtypeStruct((NROWS * NCOLS,), jnp.float32),
        mesh=mesh,
        scratch_types=[
            pltpu.VMEM((NCOLS,), jnp.float32),
            pltpu.VMEM((NCOLS,), jnp.float32),
        ],
        compiler_params=pltpu.CompilerParams(needs_layout_passes=False),
    )
    return f(al_flat, sc_flat)


_RB = 32


def _tc_loss_body(y_ref, out_ref):
    i = pl.program_id(0)
    y = y_ref[...]
    s = jnp.where(jnp.isnan(y) | jnp.isinf(y), 0.0, y)
    s = jnp.clip(s, -50.0, 50.0)
    m = jnp.max(s, axis=1, keepdims=True)
    e = jnp.exp(s - m).reshape(_RB * 64, 128)
    li = lax.broadcasted_iota(jnp.int32, (128, 128), 0)
    lj = lax.broadcasted_iota(jnp.int32, (128, 128), 1)
    tl = (li >= lj).astype(jnp.float32)
    w = lax.dot_general(
        e, tl, (((1,), (0,)), ((), ())),
        preferred_element_type=jnp.float32,
        precision=lax.Precision.HIGHEST,
    ).reshape(_RB, 64, 128)
    bs = jnp.sum(e.reshape(_RB, 64, 128), axis=2)
    bi = lax.broadcasted_iota(jnp.int32, (64, 64), 0)
    bj = lax.broadcasted_iota(jnp.int32, (64, 64), 1)
    tb = (bi > bj).astype(jnp.float32)
    sb = lax.dot_general(
        bs, tb, (((1,), (0,)), ((), ())),
        preferred_element_type=jnp.float32,
        precision=lax.Precision.HIGHEST,
    )
    suf = w + sb[:, :, None]
    logc = jnp.log(suf + EPS) + m[:, :, None]
    part = (jnp.sum(logc) - jnp.sum(s)) / (NROWS * NCOLS)

    @pl.when(i == 0)
    def _():
        out_ref[...] = jnp.zeros((1, 1), jnp.float32)

    out_ref[...] += jnp.reshape(part, (1, 1))


def _tc_loss(sorted_scores):
    out = pl.pallas_call(
        _tc_loss_body,
        grid=(NROWS // _RB,),
        in_specs=[pl.BlockSpec((_RB, NCOLS), lambda i: (i, 0))],
        out_specs=pl.BlockSpec((1, 1), lambda i: (0, 0)),
        out_shape=jax.ShapeDtypeStruct((1, 1), jnp.float32),
    )(sorted_scores)
    return out[0, 0]


def kernel(scores, auxiliary_labels):
    s = scores.astype(jnp.float32).reshape(-1)
    al = auxiliary_labels.astype(jnp.float32).reshape(-1)
    sorted_flat = _sc_sort(al, s)
    return _tc_loss(sorted_flat.reshape(NROWS, NCOLS))


# speedup vs baseline: 3.7415x; 1.7419x over previous
import jax
import jax.numpy as jnp
from jax import lax
from jax.experimental import pallas as pl
from jax.experimental.pallas import tpu as pltpu
from jax.experimental.pallas import tpu_sc as plsc

NROWS = 128
NCOLS = 8192
LANES = 16
NVREG = NCOLS // LANES
NC = 2
NS = 16
NW = NC * NS
ROWS_PER_W = NROWS // NW
EPS = 1e-10


def _vget(ref, vi):
    return ref[pl.ds(vi * LANES, LANES)]


def _vput(ref, vi, x):
    ref[pl.ds(vi * LANES, LANES)] = x


def _cmpx(rk, rv, a, b):
    ka, kb = rk[a], rk[b]
    va, vb = rv[a], rv[b]
    cm = ka <= kb
    rk[a] = jnp.where(cm, ka, kb)
    rv[a] = jnp.where(cm, va, vb)
    rk[b] = jnp.where(cm, kb, ka)
    rv[b] = jnp.where(cm, vb, va)


def _cmpx_reflect(rk, rv, a, b):
    ka, va = rk[a], rv[a]
    kb = lax.rev(rk[b], (0,))
    vb = lax.rev(rv[b], (0,))
    cm = ka <= kb
    rk[a] = jnp.where(cm, ka, kb)
    rv[a] = jnp.where(cm, va, vb)
    rk[b] = lax.rev(jnp.where(cm, kb, ka), (0,))
    rv[b] = lax.rev(jnp.where(cm, vb, va), (0,))


def _vsort_all(rk, rv):
    for j in range(8):
        rk[j], rv[j] = plsc.sort_key_val(rk[j], rv[j])


def _finish_in_regs(rk, rv):
    for a, b in ((0, 4), (1, 5), (2, 6), (3, 7)):
        _cmpx(rk, rv, a, b)
    for a, b in ((0, 2), (1, 3), (4, 6), (5, 7)):
        _cmpx(rk, rv, a, b)
    for a, b in ((0, 1), (2, 3), (4, 5), (6, 7)):
        _cmpx(rk, rv, a, b)
    _vsort_all(rk, rv)


def _sc_sort_body(al_hbm, sc_hbm, out_hbm, key_v, val_v):
    wid = lax.axis_index("s") * NC + lax.axis_index("c")

    for rr in range(ROWS_PER_W):
        base = (wid * ROWS_PER_W + rr) * NCOLS
        pltpu.sync_copy(al_hbm.at[pl.ds(base, NCOLS)], key_v)
        pltpu.sync_copy(sc_hbm.at[pl.ds(base, NCOLS)], val_v)

        def pass_a(g):
            b8 = g * 8
            rk, rv = [], []
            for j in range(8):
                a = _vget(key_v, b8 + j)
                bad = (a != a) | (jnp.abs(a) == jnp.inf)
                rk.append(-jnp.where(bad, 0.0, a))
                rv.append(_vget(val_v, b8 + j))
            _vsort_all(rk, rv)
            for a, b in ((0, 1), (2, 3), (4, 5), (6, 7)):
                _cmpx_reflect(rk, rv, a, b)
            _vsort_all(rk, rv)
            for a, b in ((0, 3), (1, 2), (4, 7), (5, 6)):
                _cmpx_reflect(rk, rv, a, b)
            for a, b in ((0, 1), (2, 3), (4, 5), (6, 7)):
                _cmpx(rk, rv, a, b)
            _vsort_all(rk, rv)
            for a, b in ((0, 7), (1, 6), (2, 5), (3, 4)):
                _cmpx_reflect(rk, rv, a, b)
            for a, b in ((0, 2), (1, 3), (4, 6), (5, 7)):
                _cmpx(rk, rv, a, b)
            for a, b in ((0, 1), (2, 3), (4, 5), (6, 7)):
                _cmpx(rk, rv, a, b)
            _vsort_all(rk, rv)
            for j in range(8):
                _vput(key_v, b8 + j, rk[j])
                _vput(val_v, b8 + j, rv[j])

        plsc.parallel_loop(0, NVREG // 8)(pass_a)

        for nv in (16, 32, 64, 128, 256, 512):
            half = nv // 2

            def reflect_body(p, nv=nv, half=half):
                blk = p // half
                j = p - blk * half
                a_i = blk * nv + j
                b_i = blk * nv + (nv - 1 - j)
                ka = _vget(key_v, a_i)
                va = _vget(val_v, a_i)
                kb = lax.rev(_vget(key_v, b_i), (0,))
                vb = lax.rev(_vget(val_v, b_i), (0,))
                cm = ka <= kb
                _vput(key_v, a_i, jnp.where(cm, ka, kb))
                _vput(val_v, a_i, jnp.where(cm, va, vb))
                _vput(key_v, b_i, lax.rev(jnp.where(cm, kb, ka), (0,)))
                _vput(val_v, b_i, lax.rev(jnp.where(cm, vb, va), (0,)))

            plsc.parallel_loop(0, NVREG // 2)(reflect_body)

            d = half // 2
            while d >= 8:

                def halv_body(p, d=d):
                    q = p // d
                    a_i = q * (2 * d) + (p - q * d)
                    b_i = a_i + d
                    ka = _vget(key_v, a_i)
                    va = _vget(val_v, a_i)
                    kb = _vget(key_v, b_i)
                    vb = _vget(val_v, b_i)
                    cm = ka <= kb
                    _vput(key_v, a_i, jnp.where(cm, ka, kb))
                    _vput(val_v, a_i, jnp.where(cm, va, vb))
                    _vput(key_v, b_i, jnp.where(cm, kb, ka))
                    _vput(val_v, b_i, jnp.where(cm, vb, va))

                plsc.parallel_loop(0, NVREG // 2)(halv_body)
                d //= 2

            def finish_body(g):
                b8 = g * 8
                rk = [_vget(key_v, b8 + j) for j in range(8)]
                rv = [_vget(val_v, b8 + j) for j in range(8)]
                _finish_in_regs(rk, rv)
                for j in range(8):
                    _vput(key_v, b8 + j, rk[j])
                    _vput(val_v, b8 + j, rv[j])

            plsc.parallel_loop(0, NVREG // 8)(finish_body)

        pltpu.sync_copy(val_v, out_hbm.at[pl.ds(base, NCOLS)])


def _sc_sort(al_flat, sc_flat):
    mesh = plsc.VectorSubcoreMesh(core_axis_name="c", subcore_axis_name="s")
    f = pl.kernel(
        _sc_sort_body,
        out_type=jax.ShapeDtypeStruct((NROWS * NCOLS,), jnp.float32),
        mesh=mesh,
        scratch_types=[
            pltpu.VMEM((NCOLS,), jnp.float32),
            pltpu.VMEM((NCOLS,), jnp.float32),
        ],
        compiler_params=pltpu.CompilerParams(needs_layout_passes=False),
    )
    return f(al_flat, sc_flat)


_RB = 32


def _tc_loss_body(y_ref, out_ref):
    i = pl.program_id(0)
    y = y_ref[...]
    s = jnp.where(jnp.isnan(y) | jnp.isinf(y), 0.0, y)
    s = jnp.clip(s, -50.0, 50.0)
    m = jnp.max(s, axis=1, keepdims=True)
    e = jnp.exp(s - m).reshape(_RB * 64, 128)
    li = lax.broadcasted_iota(jnp.int32, (128, 128), 0)
    lj = lax.broadcasted_iota(jnp.int32, (128, 128), 1)
    tl = (li >= lj).astype(jnp.float32)
    w = lax.dot_general(
        e, tl, (((1,), (0,)), ((), ())),
        preferred_element_type=jnp.float32,
        precision=lax.Precision.HIGHEST,
    ).reshape(_RB, 64, 128)
    bs = jnp.sum(e.reshape(_RB, 64, 128), axis=2)
    bi = lax.broadcasted_iota(jnp.int32, (64, 64), 0)
    bj = lax.broadcasted_iota(jnp.int32, (64, 64), 1)
    tb = (bi > bj).astype(jnp.float32)
    sb = lax.dot_general(
        bs, tb, (((1,), (0,)), ((), ())),
        preferred_element_type=jnp.float32,
        precision=lax.Precision.HIGHEST,
    )
    suf = w + sb[:, :, None]
    logc = jnp.log(suf + EPS) + m[:, :, None]
    part = (jnp.sum(logc) - jnp.sum(s)) / (NROWS * NCOLS)

    @pl.when(i == 0)
    def _():
        out_ref[...] = jnp.zeros((1, 1), jnp.float32)

    out_ref[...] += jnp.reshape(part, (1, 1))


def _tc_loss(sorted_scores):
    out = pl.pallas_call(
        _tc_loss_body,
        grid=(NROWS // _RB,),
        in_specs=[pl.BlockSpec((_RB, NCOLS), lambda i: (i, 0))],
        out_specs=pl.BlockSpec((1, 1), lambda i: (0, 0)),
        out_shape=jax.ShapeDtypeStruct((1, 1), jnp.float32),
    )(sorted_scores)
    return out[0, 0]


def kernel(scores, auxiliary_labels):
    s = scores.astype(jnp.float32).reshape(-1)
    al = auxiliary_labels.astype(jnp.float32).reshape(-1)
    sorted_flat = _sc_sort(al, s)
    return _tc_loss(sorted_flat.reshape(NROWS, NCOLS))


# speedup vs baseline: 4.4848x; 1.1987x over previous
import jax
import jax.numpy as jnp
from jax import lax
from jax.experimental import pallas as pl
from jax.experimental.pallas import tpu as pltpu
from jax.experimental.pallas import tpu_sc as plsc

NROWS = 128
NCOLS = 8192
LANES = 16
NVREG = NCOLS // LANES
NC = 2
NS = 16
NW = NC * NS
ROWS_PER_W = NROWS // NW
EPS = 1e-10


def _vget(ref, vi):
    return ref[pl.ds(vi * LANES, LANES)]


def _vput(ref, vi, x):
    ref[pl.ds(vi * LANES, LANES)] = x


def _cmpx(rk, rv, a, b):
    ka, kb = rk[a], rk[b]
    va, vb = rv[a], rv[b]
    cm = ka <= kb
    rk[a] = jnp.where(cm, ka, kb)
    rv[a] = jnp.where(cm, va, vb)
    rk[b] = jnp.where(cm, kb, ka)
    rv[b] = jnp.where(cm, vb, va)


def _cmpx_reflect(rk, rv, a, b):
    ka, va = rk[a], rv[a]
    kb = lax.rev(rk[b], (0,))
    vb = lax.rev(rv[b], (0,))
    cm = ka <= kb
    rk[a] = jnp.where(cm, ka, kb)
    rv[a] = jnp.where(cm, va, vb)
    rk[b] = lax.rev(jnp.where(cm, kb, ka), (0,))
    rv[b] = lax.rev(jnp.where(cm, vb, va), (0,))


def _vsort_all(rk, rv):
    for j in range(8):
        rk[j], rv[j] = plsc.sort_key_val(rk[j], rv[j])


def _finish_in_regs(rk, rv):
    for a, b in ((0, 4), (1, 5), (2, 6), (3, 7)):
        _cmpx(rk, rv, a, b)
    for a, b in ((0, 2), (1, 3), (4, 6), (5, 7)):
        _cmpx(rk, rv, a, b)
    for a, b in ((0, 1), (2, 3), (4, 5), (6, 7)):
        _cmpx(rk, rv, a, b)
    _vsort_all(rk, rv)


def _sc_sort_body(al_hbm, sc_hbm, out_hbm, key_v, val_v):
    wid = lax.axis_index("s") * NC + lax.axis_index("c")

    def row_body(rr, _):
        base = (wid * ROWS_PER_W + rr) * NCOLS
        pltpu.sync_copy(al_hbm.at[pl.ds(base, NCOLS)], key_v)
        pltpu.sync_copy(sc_hbm.at[pl.ds(base, NCOLS)], val_v)

        def pass_a(g):
            b8 = g * 8
            rk, rv = [], []
            for j in range(8):
                a = _vget(key_v, b8 + j)
                bad = (a != a) | (jnp.abs(a) == jnp.inf)
                rk.append(-jnp.where(bad, 0.0, a))
                rv.append(_vget(val_v, b8 + j))
            _vsort_all(rk, rv)
            for a, b in ((0, 1), (2, 3), (4, 5), (6, 7)):
                _cmpx_reflect(rk, rv, a, b)
            _vsort_all(rk, rv)
            for a, b in ((0, 3), (1, 2), (4, 7), (5, 6)):
                _cmpx_reflect(rk, rv, a, b)
            for a, b in ((0, 1), (2, 3), (4, 5), (6, 7)):
                _cmpx(rk, rv, a, b)
            _vsort_all(rk, rv)
            for a, b in ((0, 7), (1, 6), (2, 5), (3, 4)):
                _cmpx_reflect(rk, rv, a, b)
            for a, b in ((0, 2), (1, 3), (4, 6), (5, 7)):
                _cmpx(rk, rv, a, b)
            for a, b in ((0, 1), (2, 3), (4, 5), (6, 7)):
                _cmpx(rk, rv, a, b)
            _vsort_all(rk, rv)
            for j in range(8):
                _vput(key_v, b8 + j, rk[j])
                _vput(val_v, b8 + j, rv[j])

        plsc.parallel_loop(0, NVREG // 8)(pass_a)

        for nv in (16, 32, 64, 128, 256, 512):
            half = nv // 2

            def reflect_body(q, nv=nv, half=half):
                p4 = q * 4
                blk = p4 // half
                j4 = p4 - blk * half
                a_i = blk * nv + j4
                b_i = blk * nv + (nv - 4 - j4)
                ak = [_vget(key_v, a_i + t) for t in range(4)]
                av = [_vget(val_v, a_i + t) for t in range(4)]
                bk = [_vget(key_v, b_i + t) for t in range(4)]
                bv = [_vget(val_v, b_i + t) for t in range(4)]
                for t in range(4):
                    u = 3 - t
                    ka, va = ak[t], av[t]
                    kb = lax.rev(bk[u], (0,))
                    vb = lax.rev(bv[u], (0,))
                    cm = ka <= kb
                    ak[t] = jnp.where(cm, ka, kb)
                    av[t] = jnp.where(cm, va, vb)
                    bk[u] = lax.rev(jnp.where(cm, kb, ka), (0,))
                    bv[u] = lax.rev(jnp.where(cm, vb, va), (0,))
                for t in range(4):
                    _vput(key_v, a_i + t, ak[t])
                    _vput(val_v, a_i + t, av[t])
                    _vput(key_v, b_i + t, bk[t])
                    _vput(val_v, b_i + t, bv[t])

            plsc.parallel_loop(0, NVREG // 8)(reflect_body)

            d = half // 2
            while d >= 8:

                def halv_body(q, d=d):
                    p4 = q * 4
                    blk = p4 // d
                    a_i = blk * (2 * d) + (p4 - blk * d)
                    b_i = a_i + d
                    ak = [_vget(key_v, a_i + t) for t in range(4)]
                    av = [_vget(val_v, a_i + t) for t in range(4)]
                    bk = [_vget(key_v, b_i + t) for t in range(4)]
                    bv = [_vget(val_v, b_i + t) for t in range(4)]
                    for t in range(4):
                        ka, kb = ak[t], bk[t]
                        va, vb = av[t], bv[t]
                        cm = ka <= kb
                        ak[t] = jnp.where(cm, ka, kb)
                        av[t] = jnp.where(cm, va, vb)
                        bk[t] = jnp.where(cm, kb, ka)
                        bv[t] = jnp.where(cm, vb, va)
                    for t in range(4):
                        _vput(key_v, a_i + t, ak[t])
                        _vput(val_v, a_i + t, av[t])
                        _vput(key_v, b_i + t, bk[t])
                        _vput(val_v, b_i + t, bv[t])

                plsc.parallel_loop(0, NVREG // 8)(halv_body)
                d //= 2

            def finish_body(g):
                b8 = g * 8
                rk = [_vget(key_v, b8 + j) for j in range(8)]
                rv = [_vget(val_v, b8 + j) for j in range(8)]
                _finish_in_regs(rk, rv)
                for j in range(8):
                    _vput(key_v, b8 + j, rk[j])
                    _vput(val_v, b8 + j, rv[j])

            plsc.parallel_loop(0, NVREG // 8)(finish_body)

        pltpu.sync_copy(val_v, out_hbm.at[pl.ds(base, NCOLS)])
        return 0

    lax.fori_loop(0, ROWS_PER_W, row_body, 0)


def _sc_sort(al_flat, sc_flat):
    mesh = plsc.VectorSubcoreMesh(core_axis_name="c", subcore_axis_name="s")
    f = pl.kernel(
        _sc_sort_body,
        out_type=jax.ShapeDtypeStruct((NROWS * NCOLS,), jnp.float32),
        mesh=mesh,
        scratch_types=[
            pltpu.VMEM((NCOLS,), jnp.float32),
            pltpu.VMEM((NCOLS,), jnp.float32),
        ],
        compiler_params=pltpu.CompilerParams(needs_layout_passes=False),
    )
    return f(al_flat, sc_flat)


_RB = 32


def _tc_loss_body(y_ref, out_ref):
    i = pl.program_id(0)
    y = y_ref[...]
    s = jnp.where(jnp.isnan(y) | jnp.isinf(y), 0.0, y)
    s = jnp.clip(s, -50.0, 50.0)
    m = jnp.max(s, axis=1, keepdims=True)
    e = jnp.exp(s - m).reshape(_RB * 64, 128)
    li = lax.broadcasted_iota(jnp.int32, (128, 128), 0)
    lj = lax.broadcasted_iota(jnp.int32, (128, 128), 1)
    tl = (li >= lj).astype(jnp.float32)
    w = lax.dot_general(
        e, tl, (((1,), (0,)), ((), ())),
        preferred_element_type=jnp.float32,
        precision=lax.Precision.HIGHEST,
    ).reshape(_RB, 64, 128)
    bs = jnp.sum(e.reshape(_RB, 64, 128), axis=2)
    bi = lax.broadcasted_iota(jnp.int32, (64, 64), 0)
    bj = lax.broadcasted_iota(jnp.int32, (64, 64), 1)
    tb = (bi > bj).astype(jnp.float32)
    sb = lax.dot_general(
        bs, tb, (((1,), (0,)), ((), ())),
        preferred_element_type=jnp.float32,
        precision=lax.Precision.HIGHEST,
    )
    suf = w + sb[:, :, None]
    logc = jnp.log(suf + EPS) + m[:, :, None]
    part = (jnp.sum(logc) - jnp.sum(s)) / (NROWS * NCOLS)

    @pl.when(i == 0)
    def _():
        out_ref[...] = jnp.zeros((1, 1), jnp.float32)

    out_ref[...] += jnp.reshape(part, (1, 1))


def _tc_loss(sorted_scores):
    out = pl.pallas_call(
        _tc_loss_body,
        grid=(NROWS // _RB,),
        in_specs=[pl.BlockSpec((_RB, NCOLS), lambda i: (i, 0))],
        out_specs=pl.BlockSpec((1, 1), lambda i: (0, 0)),
        out_shape=jax.ShapeDtypeStruct((1, 1), jnp.float32),
    )(sorted_scores)
    return out[0, 0]


def kernel(scores, auxiliary_labels):
    s = scores.astype(jnp.float32).reshape(-1)
    al = auxiliary_labels.astype(jnp.float32).reshape(-1)
    sorted_flat = _sc_sort(al, s)
    return _tc_loss(sorted_flat.reshape(NROWS, NCOLS))


# speedup vs baseline: 6.0353x; 1.3457x over previous
import jax
import jax.numpy as jnp
import numpy as np
from jax import lax
from jax.experimental import pallas as pl
from jax.experimental.pallas import tpu as pltpu
from jax.experimental.pallas import tpu_sc as plsc

NROWS = 128
NCOLS = 8192
LANES = 16
NVREG = NCOLS // LANES
NC = 2
NS = 16
NW = NC * NS
ROWS_PER_W = NROWS // NW
EPS = 1e-10

_HI_MASK = np.uint32(0xFFFFE000)
_IDX_MASK = np.int32(0x1FFF)
_SIGN = np.uint32(0x80000000)

_REFL = {
    32: [(0, 1), (2, 3), (4, 5), (6, 7), (8, 9), (10, 11), (12, 13), (14, 15)],
    64: [(0, 3), (1, 2), (4, 7), (5, 6), (8, 11), (9, 10), (12, 15), (13, 14)],
    128: [(0, 7), (1, 6), (2, 5), (3, 4), (8, 15), (9, 14), (10, 13), (11, 12)],
    256: [(0, 15), (1, 14), (2, 13), (3, 12), (4, 11), (5, 10), (6, 9), (7, 8)],
}
_D8 = [(i, i + 8) for i in range(8)]
_D4 = [(i, i + 4) for i in (0, 1, 2, 3, 8, 9, 10, 11)]
_D2 = [(i, i + 2) for i in (0, 1, 4, 5, 8, 9, 12, 13)]
_D1 = [(2 * i, 2 * i + 1) for i in range(8)]


def _vget(ref, vi):
    return ref[pl.ds(vi * LANES, LANES)]


def _vput(ref, vi, x):
    ref[pl.ds(vi * LANES, LANES)] = x


def _cmpx(rk, a, b):
    lo = jnp.minimum(rk[a], rk[b])
    hi = jnp.maximum(rk[a], rk[b])
    rk[a], rk[b] = lo, hi


def _cmpx_reflect(rk, a, b):
    ka = rk[a]
    kb = lax.rev(rk[b], (0,))
    rk[a] = jnp.minimum(ka, kb)
    rk[b] = lax.rev(jnp.maximum(ka, kb), (0,))


def _vsort_all(rk):
    for j in range(16):
        rk[j] = jnp.sort(rk[j])


def _sc_sort_body(al_hbm, sc_hbm, out_hbm, key_v, val_v, aux_v):
    wid = lax.axis_index("s") * NC + lax.axis_index("c")
    lane_iota = plsc.bitcast(lax.iota(jnp.int32, 16), jnp.uint32)

    def row_body(rr, _):
        base = (wid * ROWS_PER_W + rr) * NCOLS
        pltpu.sync_copy(al_hbm.at[pl.ds(base, NCOLS)], aux_v)
        pltpu.sync_copy(sc_hbm.at[pl.ds(base, NCOLS)], val_v)

        def pass_a(g):
            b16 = g * 16
            rk = []
            for j in range(16):
                a = _vget(aux_v, b16 + j)
                bad = (a != a) | (jnp.abs(a) == jnp.inf)
                a = jnp.where(bad, 0.0, a)
                u = plsc.bitcast(a, jnp.uint32)
                elem = lane_iota + ((b16 + j) * LANES).astype(jnp.uint32)
                ku = ((~u) & _HI_MASK) | elem
                rk.append(plsc.bitcast(ku ^ _SIGN, jnp.int32))
            _vsort_all(rk)
            for kblk in (32, 64, 128, 256):
                for a, b in _REFL[kblk]:
                    _cmpx_reflect(rk, a, b)
                if kblk >= 256:
                    for a, b in _D4:
                        _cmpx(rk, a, b)
                if kblk >= 128:
                    for a, b in _D2:
                        _cmpx(rk, a, b)
                if kblk >= 64:
                    for a, b in _D1:
                        _cmpx(rk, a, b)
                _vsort_all(rk)
            for j in range(16):
                _vput(key_v, b16 + j, rk[j])

        plsc.parallel_loop(0, NVREG // 16)(pass_a)

        for nv in (32, 64, 128, 256, 512):
            half = nv // 2

            def reflect_body(q, nv=nv, half=half):
                p8 = q * 8
                blk = p8 // half
                j8 = p8 - blk * half
                a_i = blk * nv + j8
                b_i = blk * nv + (nv - 8 - j8)
                ak = [_vget(key_v, a_i + t) for t in range(8)]
                bk = [_vget(key_v, b_i + t) for t in range(8)]
                for t in range(8):
                    u = 7 - t
                    ka = ak[t]
                    kb = lax.rev(bk[u], (0,))
                    ak[t] = jnp.minimum(ka, kb)
                    bk[u] = lax.rev(jnp.maximum(ka, kb), (0,))
                for t in range(8):
                    _vput(key_v, a_i + t, ak[t])
                    _vput(key_v, b_i + t, bk[t])

            plsc.parallel_loop(0, NVREG // 16)(reflect_body)

            d = half // 2
            while d >= 16:

                def halv_body(q, d=d):
                    p8 = q * 8
                    blk = p8 // d
                    a_i = blk * (2 * d) + (p8 - blk * d)
                    b_i = a_i + d
                    ak = [_vget(key_v, a_i + t) for t in range(8)]
                    bk = [_vget(key_v, b_i + t) for t in range(8)]
                    for t in range(8):
                        lo = jnp.minimum(ak[t], bk[t])
                        hi = jnp.maximum(ak[t], bk[t])
                        ak[t], bk[t] = lo, hi
                    for t in range(8):
                        _vput(key_v, a_i + t, ak[t])
                        _vput(key_v, b_i + t, bk[t])

                plsc.parallel_loop(0, NVREG // 16)(halv_body)
                d //= 2

            last = nv == 512

            def finish_body(g, last=last):
                b16 = g * 16
                rk = [_vget(key_v, b16 + j) for j in range(16)]
                for a, b in _D8 + _D4 + _D2 + _D1:
                    _cmpx(rk, a, b)
                _vsort_all(rk)
                if last:
                    for j in range(16):
                        idx = rk[j] & _IDX_MASK
                        _vput(aux_v, b16 + j, plsc.load_gather(val_v, [idx]))
                else:
                    for j in range(16):
                        _vput(key_v, b16 + j, rk[j])

            plsc.parallel_loop(0, NVREG // 16)(finish_body)

        pltpu.sync_copy(aux_v, out_hbm.at[pl.ds(base, NCOLS)])
        return 0

    lax.fori_loop(0, ROWS_PER_W, row_body, 0)


def _sc_sort(al_flat, sc_flat):
    mesh = plsc.VectorSubcoreMesh(core_axis_name="c", subcore_axis_name="s")
    f = pl.kernel(
        _sc_sort_body,
        out_type=jax.ShapeDtypeStruct((NROWS * NCOLS,), jnp.float32),
        mesh=mesh,
        scratch_types=[
            pltpu.VMEM((NCOLS,), jnp.int32),
            pltpu.VMEM((NCOLS,), jnp.float32),
            pltpu.VMEM((NCOLS,), jnp.float32),
        ],
        compiler_params=pltpu.CompilerParams(needs_layout_passes=False),
    )
    return f(al_flat, sc_flat)


_RB = 32


def _tc_loss_body(y_ref, out_ref):
    i = pl.program_id(0)
    y = y_ref[...]
    s = jnp.where(jnp.isnan(y) | jnp.isinf(y), 0.0, y)
    s = jnp.clip(s, -50.0, 50.0)
    m = jnp.max(s, axis=1, keepdims=True)
    e = jnp.exp(s - m).reshape(_RB * 64, 128)
    li = lax.broadcasted_iota(jnp.int32, (128, 128), 0)
    lj = lax.broadcasted_iota(jnp.int32, (128, 128), 1)
    tl = (li >= lj).astype(jnp.float32)
    w = lax.dot_general(
        e, tl, (((1,), (0,)), ((), ())),
        preferred_element_type=jnp.float32,
        precision=lax.Precision.HIGHEST,
    ).reshape(_RB, 64, 128)
    bs = jnp.sum(e.reshape(_RB, 64, 128), axis=2)
    bi = lax.broadcasted_iota(jnp.int32, (64, 64), 0)
    bj = lax.broadcasted_iota(jnp.int32, (64, 64), 1)
    tb = (bi > bj).astype(jnp.float32)
    sb = lax.dot_general(
        bs, tb, (((1,), (0,)), ((), ())),
        preferred_element_type=jnp.float32,
        precision=lax.Precision.HIGHEST,
    )
    suf = w + sb[:, :, None]
    logc = jnp.log(suf + EPS) + m[:, :, None]
    part = (jnp.sum(logc) - jnp.sum(s)) / (NROWS * NCOLS)

    @pl.when(i == 0)
    def _():
        out_ref[...] = jnp.zeros((1, 1), jnp.float32)

    out_ref[...] += jnp.reshape(part, (1, 1))


def _tc_loss(sorted_scores):
    out = pl.pallas_call(
        _tc_loss_body,
        grid=(NROWS // _RB,),
        in_specs=[pl.BlockSpec((_RB, NCOLS), lambda i: (i, 0))],
        out_specs=pl.BlockSpec((1, 1), lambda i: (0, 0)),
        out_shape=jax.ShapeDtypeStruct((1, 1), jnp.float32),
    )(sorted_scores)
    return out[0, 0]


def kernel(scores, auxiliary_labels):
    s = scores.astype(jnp.float32).reshape(-1)
    al = auxiliary_labels.astype(jnp.float32).reshape(-1)
    sorted_flat = _sc_sort(al, s)
    return _tc_loss(sorted_flat.reshape(NROWS, NCOLS))
